# baseline (speedup 1.0000x reference)
import functools

import jax
import jax.numpy as jnp
from jax import lax
from jax.experimental import pallas as pl
from jax.experimental.pallas import tpu as pltpu
from jax.experimental.pallas import tpu_sc as plsc

N = 16384
WIN = 5
EMB = 64
NCLASS = 9

_NUM_IDX = N * WIN

_NC = 2
_NS = 16
_NW = _NC * _NS
_PER_W = _NUM_IDX // _NW
_CHUNK = 512
_NCHUNK = _PER_W // _CHUNK


def _sc_gather(idx, table):
    mesh = plsc.VectorSubcoreMesh(core_axis_name="c", subcore_axis_name="s")

    @functools.partial(
        pl.kernel,
        out_type=jax.ShapeDtypeStruct((_NUM_IDX, EMB), jnp.float32),
        mesh=mesh,
        scratch_types=[
            pltpu.VMEM((_PER_W,), jnp.int32),
            pltpu.VMEM((_CHUNK, EMB), jnp.float32),
            pltpu.VMEM((_CHUNK, EMB), jnp.float32),
            pltpu.SemaphoreType.DMA,
            pltpu.SemaphoreType.DMA,
        ],
        compiler_params=pltpu.CompilerParams(use_tc_tiling_on_sc=False),
    )
    def gather_kernel(idx_hbm, table_hbm, out_hbm, idx_v, rows0, rows1, sem0, sem1):
        wid = lax.axis_index("s") * _NC + lax.axis_index("c")
        base = wid * _PER_W
        pltpu.sync_copy(idx_hbm.at[pl.ds(base, _PER_W)], idx_v)
        bufs = (rows0, rows1)
        sems = (sem0, sem1)
        copies = [pltpu.async_copy(table_hbm.at[idx_v.at[pl.ds(0, _CHUNK)]], rows0, sem0)]
        for c in range(_NCHUNK):
            if c + 1 < _NCHUNK:
                copies.append(pltpu.async_copy(
                    table_hbm.at[idx_v.at[pl.ds((c + 1) * _CHUNK, _CHUNK)]],
                    bufs[(c + 1) % 2],
                    sems[(c + 1) % 2],
                ))
            copies[c].wait()
            pltpu.sync_copy(bufs[c % 2], out_hbm.at[pl.ds(base + c * _CHUNK, _CHUNK)])

    return gather_kernel(idx, table)


def _tc_linear(flat, wt, b2d):
    bn = 4096

    def mm_kernel(flat_ref, wt_ref, b_ref, out_ref):
        out_ref[...] = (
            jnp.dot(flat_ref[...], wt_ref[...], preferred_element_type=jnp.float32)
            + b_ref[...]
        )

    return pl.pallas_call(
        mm_kernel,
        grid=(N // bn,),
        in_specs=[
            pl.BlockSpec((bn, WIN * EMB), lambda i: (i, 0)),
            pl.BlockSpec((WIN * EMB, NCLASS), lambda i: (0, 0)),
            pl.BlockSpec((1, NCLASS), lambda i: (0, 0)),
        ],
        out_specs=pl.BlockSpec((bn, NCLASS), lambda i: (i, 0)),
        out_shape=jax.ShapeDtypeStruct((N, NCLASS), jnp.float32),
    )(flat, wt, b2d)


def kernel(x, table, W, b):
    idx = x.reshape(-1).astype(jnp.int32)
    rows = _sc_gather(idx, table)
    flat = rows.reshape(N, WIN * EMB)
    out = _tc_linear(flat, W.T, b.reshape(1, NCLASS))
    return out
